# baseline (speedup 1.0000x reference)
import functools

import jax
import jax.numpy as jnp
from jax import lax
from jax.experimental import pallas as pl
from jax.experimental.pallas import tpu as pltpu
from jax.experimental.pallas import tpu_sc as plsc

VOCAB_SIZE = 100000
EMBED_DIM = 128
MAXLEN = 200
BATCH = 1024

NUM_CORES = 2
NUM_SUBCORES = 16
NUM_WORKERS = NUM_CORES * NUM_SUBCORES

SUBGATHER = 100
SUBS = MAXLEN // SUBGATHER
N = BATCH // NUM_WORKERS
LANES = 16
VECS_PER_ROW = EMBED_DIM // LANES


def _emb_kernel(idx_hbm, token_hbm, pos_hbm, out_hbm,
                pos_v, idx0, idx1, rows0, rows1,
                isem0, isem1, gsem0, gsem1, ssem0, ssem1):
    wid = lax.axis_index("s") * NUM_CORES + lax.axis_index("c")
    base = wid * N
    idx_b = (idx0, idx1)
    rows_b = (rows0, rows1)
    isem_b = (isem0, isem1)
    gsem_b = (gsem0, gsem1)
    ssem_b = (ssem0, ssem1)

    pltpu.sync_copy(pos_hbm, pos_v)

    def start_gathers(p, j):
        return [
            pltpu.async_copy(
                token_hbm.at[idx_b[p].at[h]],
                rows_b[p].at[pl.ds(h * SUBGATHER, SUBGATHER)],
                gsem_b[p],
            )
            for h in range(SUBS)
        ]

    def add_pos(p):
        def row_body(r, c2):
            for v in range(VECS_PER_ROW):
                sl = pl.ds(v * LANES, LANES)
                plsc.addupdate(rows_b[p].at[r, sl], pos_v[r, sl])
            return c2

        lax.fori_loop(0, MAXLEN, row_body, 0, unroll=4)

    pltpu.sync_copy(idx_hbm.at[base], idx0)
    g = {0: start_gathers(0, 0)}
    i = {1: pltpu.async_copy(idx_hbm.at[base + 1], idx1, isem1)}
    s = {}
    for j in range(N):
        p = j & 1
        q = p ^ 1
        if j + 1 < N:
            if j >= 1:
                s[j - 1].wait()
            i[j + 1].wait()
            g[j + 1] = start_gathers(q, j + 1)
        for d in g[j]:
            d.wait()
        if j + 2 < N:
            i[j + 2] = pltpu.async_copy(idx_hbm.at[base + j + 2], idx_b[p], isem_b[p])
        add_pos(p)
        s[j] = pltpu.async_copy(
            rows_b[p], out_hbm.at[pl.ds((base + j) * MAXLEN, MAXLEN)], ssem_b[p]
        )
    s[N - 2].wait()
    s[N - 1].wait()


@functools.partial(jax.jit, static_argnames=())
def kernel(inputs, token_table, pos_table):
    idx = inputs.reshape(BATCH, SUBS, SUBGATHER).astype(jnp.int32)
    mesh = plsc.VectorSubcoreMesh(core_axis_name="c", subcore_axis_name="s")
    out = pl.kernel(
        _emb_kernel,
        mesh=mesh,
        out_type=jax.ShapeDtypeStruct((BATCH * MAXLEN, EMBED_DIM), jnp.float32),
        scratch_types=[
            pltpu.VMEM((MAXLEN, EMBED_DIM), jnp.float32),
            pltpu.VMEM((SUBS, SUBGATHER), jnp.int32),
            pltpu.VMEM((SUBS, SUBGATHER), jnp.int32),
            pltpu.VMEM((MAXLEN, EMBED_DIM), jnp.float32),
            pltpu.VMEM((MAXLEN, EMBED_DIM), jnp.float32),
            pltpu.SemaphoreType.DMA,
            pltpu.SemaphoreType.DMA,
            pltpu.SemaphoreType.DMA,
            pltpu.SemaphoreType.DMA,
            pltpu.SemaphoreType.DMA,
            pltpu.SemaphoreType.DMA,
        ],
    )(idx, token_table, pos_table)
    return out.reshape(BATCH, MAXLEN, EMBED_DIM)


# speedup vs baseline: 1.0783x; 1.0783x over previous
import functools

import jax
import jax.numpy as jnp
from jax import lax
from jax.experimental import pallas as pl
from jax.experimental.pallas import tpu as pltpu
from jax.experimental.pallas import tpu_sc as plsc

VOCAB_SIZE = 100000
EMBED_DIM = 128
MAXLEN = 200
BATCH = 1024

NUM_CORES = 2
NUM_SUBCORES = 16
NUM_WORKERS = NUM_CORES * NUM_SUBCORES

SUBGATHER = 100
SUBS = MAXLEN // SUBGATHER
N = BATCH // NUM_WORKERS
LANES = 16
VECS_PER_ROW = EMBED_DIM // LANES


def _emb_kernel(idx_hbm, token_hbm, pos_hbm, out_hbm,
                pos_v, idx0, idx1, rows0, rows1,
                isem0, isem1, gsem0, gsem1, ssem0, ssem1):
    wid = lax.axis_index("s") * NUM_CORES + lax.axis_index("c")
    base = wid * N
    idx_b = (idx0, idx1)
    rows_b = (rows0, rows1)
    isem_b = (isem0, isem1)
    gsem_b = (gsem0, gsem1)
    ssem_b = (ssem0, ssem1)

    pltpu.sync_copy(pos_hbm, pos_v)

    def start_gathers(p, j):
        for h in range(SUBS):
            pltpu.async_copy(
                token_hbm.at[idx_b[p].at[h]],
                rows_b[p].at[pl.ds(h * SUBGATHER, SUBGATHER)],
                gsem_b[p],
            )

    def wait_gathers(p):
        for h in range(SUBS):
            pltpu.make_async_copy(
                token_hbm.at[idx_b[p].at[h]],
                rows_b[p].at[pl.ds(h * SUBGATHER, SUBGATHER)],
                gsem_b[p],
            ).wait()

    def wait_idx(p):
        pltpu.make_async_copy(idx_hbm.at[base], idx_b[p], isem_b[p]).wait()

    def wait_store(p):
        pltpu.make_async_copy(
            rows_b[p], out_hbm.at[pl.ds(0, MAXLEN)], ssem_b[p]
        ).wait()

    def add_pos(p):
        def row_body(r, c2):
            for v in range(VECS_PER_ROW):
                sl = pl.ds(v * LANES, LANES)
                plsc.addupdate(rows_b[p].at[r, sl], pos_v[r, sl])
            return c2

        lax.fori_loop(0, MAXLEN, row_body, 0, unroll=False)

    pltpu.sync_copy(idx_hbm.at[base], idx0)
    start_gathers(0, 0)
    pltpu.async_copy(idx_hbm.at[base + 1], idx1, isem1)

    def pair_body(i, carry):
        for p in range(2):
            q = p ^ 1
            j = 2 * i + p

            @pl.when(j >= 1)
            def _():
                wait_store(q)

            @pl.when(j + 1 < N)
            def _():
                wait_idx(q)
                start_gathers(q, j + 1)

            wait_gathers(p)

            @pl.when(j + 2 < N)
            def _():
                pltpu.async_copy(idx_hbm.at[base + j + 2], idx_b[p], isem_b[p])

            add_pos(p)
            pltpu.async_copy(
                rows_b[p], out_hbm.at[pl.ds((base + j) * MAXLEN, MAXLEN)],
                ssem_b[p],
            )
        return carry

    lax.fori_loop(0, N // 2, pair_body, 0, unroll=False)
    wait_store((N - 1) & 1)


@functools.partial(jax.jit, static_argnames=())
def kernel(inputs, token_table, pos_table):
    idx = inputs.reshape(BATCH, SUBS, SUBGATHER).astype(jnp.int32)
    mesh = plsc.VectorSubcoreMesh(core_axis_name="c", subcore_axis_name="s")
    out = pl.kernel(
        _emb_kernel,
        mesh=mesh,
        out_type=jax.ShapeDtypeStruct((BATCH * MAXLEN, EMBED_DIM), jnp.float32),
        scratch_types=[
            pltpu.VMEM((MAXLEN, EMBED_DIM), jnp.float32),
            pltpu.VMEM((SUBS, SUBGATHER), jnp.int32),
            pltpu.VMEM((SUBS, SUBGATHER), jnp.int32),
            pltpu.VMEM((MAXLEN, EMBED_DIM), jnp.float32),
            pltpu.VMEM((MAXLEN, EMBED_DIM), jnp.float32),
            pltpu.SemaphoreType.DMA,
            pltpu.SemaphoreType.DMA,
            pltpu.SemaphoreType.DMA,
            pltpu.SemaphoreType.DMA,
            pltpu.SemaphoreType.DMA,
            pltpu.SemaphoreType.DMA,
        ],
    )(idx, token_table, pos_table)
    return out.reshape(BATCH, MAXLEN, EMBED_DIM)


# speedup vs baseline: 1.2167x; 1.1284x over previous
import functools

import jax
import jax.numpy as jnp
from jax import lax
from jax.experimental import pallas as pl
from jax.experimental.pallas import tpu as pltpu
from jax.experimental.pallas import tpu_sc as plsc

VOCAB_SIZE = 100000
EMBED_DIM = 128
MAXLEN = 200
BATCH = 1024

NUM_CORES = 2
NUM_SUBCORES = 16
NUM_WORKERS = NUM_CORES * NUM_SUBCORES

SUBGATHER = 100
SUBS = MAXLEN // SUBGATHER
N = BATCH // NUM_WORKERS
LANES = 16
VECS_PER_ROW = EMBED_DIM // LANES


def _emb_kernel(idx_hbm, token_hbm, pos_hbm, out_hbm,
                pos_v, idx0, idx1, rows0, rows1,
                isem0, isem1, gsem00, gsem01, gsem10, gsem11, ssem0, ssem1):
    wid = lax.axis_index("s") * NUM_CORES + lax.axis_index("c")
    base = wid * N
    idx_b = (idx0, idx1)
    rows_b = (rows0, rows1)
    isem_b = (isem0, isem1)
    gsem_b = ((gsem00, gsem01), (gsem10, gsem11))
    ssem_b = (ssem0, ssem1)

    pltpu.sync_copy(pos_hbm, pos_v)

    def gather_half(p, h):
        return (
            token_hbm.at[idx_b[p].at[h]],
            rows_b[p].at[pl.ds(h * SUBGATHER, SUBGATHER)],
            gsem_b[p][h],
        )

    def start_gathers(p):
        for h in range(SUBS):
            pltpu.async_copy(*gather_half(p, h))

    def wait_gather_half(p, h):
        pltpu.make_async_copy(*gather_half(p, h)).wait()

    def wait_idx(p):
        pltpu.make_async_copy(idx_hbm.at[base], idx_b[p], isem_b[p]).wait()

    def wait_store(p):
        pltpu.make_async_copy(
            rows_b[p], out_hbm.at[pl.ds(0, MAXLEN)], ssem_b[p]
        ).wait()

    STORE_LO, STORE_HI = 96, MAXLEN - 96

    def add_pos_rows(p, lo, n):
        def row_body(r, c2):
            for v in range(VECS_PER_ROW):
                sl = pl.ds(v * LANES, LANES)
                plsc.addupdate(rows_b[p].at[r, sl], pos_v[r, sl])
            return c2

        lax.fori_loop(lo, lo + n, row_body, 0, unroll=False)

    def store_rows(p, j, lo, n):
        pltpu.async_copy(
            rows_b[p].at[pl.ds(lo, n)],
            out_hbm.at[pl.ds((base + j) * MAXLEN + lo, n)],
            ssem_b[p],
        )

    pltpu.sync_copy(idx_hbm.at[base], idx0)
    start_gathers(0)
    pltpu.async_copy(idx_hbm.at[base + 1], idx1, isem1)

    def pair_body(i, carry):
        for p in range(2):
            q = p ^ 1
            j = 2 * i + p

            @pl.when(j >= 1)
            def _():
                wait_store(q)

            @pl.when(j + 1 < N)
            def _():
                wait_idx(q)
                start_gathers(q)

            for h in range(SUBS):
                wait_gather_half(p, h)
                if h == SUBS - 1:
                    @pl.when(j + 2 < N)
                    def _():
                        pltpu.async_copy(
                            idx_hbm.at[base + j + 2], idx_b[p], isem_b[p]
                        )
                if h == 0:
                    add_pos_rows(p, 0, STORE_LO)
                    store_rows(p, j, 0, STORE_LO)
                else:
                    add_pos_rows(p, STORE_LO, STORE_HI)
                    store_rows(p, j, STORE_LO, STORE_HI)
        return carry

    lax.fori_loop(0, N // 2, pair_body, 0, unroll=False)
    wait_store((N - 1) & 1)


@functools.partial(jax.jit, static_argnames=())
def kernel(inputs, token_table, pos_table):
    idx = inputs.reshape(BATCH, SUBS, SUBGATHER).astype(jnp.int32)
    mesh = plsc.VectorSubcoreMesh(core_axis_name="c", subcore_axis_name="s")
    out = pl.kernel(
        _emb_kernel,
        mesh=mesh,
        out_type=jax.ShapeDtypeStruct((BATCH * MAXLEN, EMBED_DIM), jnp.float32),
        scratch_types=[
            pltpu.VMEM((MAXLEN, EMBED_DIM), jnp.float32),
            pltpu.VMEM((SUBS, SUBGATHER), jnp.int32),
            pltpu.VMEM((SUBS, SUBGATHER), jnp.int32),
            pltpu.VMEM((MAXLEN, EMBED_DIM), jnp.float32),
            pltpu.VMEM((MAXLEN, EMBED_DIM), jnp.float32),
            pltpu.SemaphoreType.DMA,
            pltpu.SemaphoreType.DMA,
            pltpu.SemaphoreType.DMA,
            pltpu.SemaphoreType.DMA,
            pltpu.SemaphoreType.DMA,
            pltpu.SemaphoreType.DMA,
            pltpu.SemaphoreType.DMA,
            pltpu.SemaphoreType.DMA,
        ],
    )(idx, token_table, pos_table)
    return out.reshape(BATCH, MAXLEN, EMBED_DIM)


# speedup vs baseline: 1.2539x; 1.0306x over previous
import functools

import jax
import jax.numpy as jnp
from jax import lax
from jax.experimental import pallas as pl
from jax.experimental.pallas import tpu as pltpu
from jax.experimental.pallas import tpu_sc as plsc

VOCAB_SIZE = 100000
EMBED_DIM = 128
MAXLEN = 200
BATCH = 1024

NUM_CORES = 2
NUM_SUBCORES = 16
NUM_WORKERS = NUM_CORES * NUM_SUBCORES

SUBGATHER = 100
SUBS = MAXLEN // SUBGATHER
N = BATCH // NUM_WORKERS
NBUF = 4
LANES = 16
VECS_PER_ROW = EMBED_DIM // LANES

STORE_LO = 96
STORE_HI = MAXLEN - STORE_LO


def _emb_kernel(idx_hbm, token_hbm, pos_hbm, out_hbm, pos_v, *rest):
    idx_b = rest[0:NBUF]
    rows_b = rest[NBUF:2 * NBUF]
    isem_b = rest[2 * NBUF:3 * NBUF]
    gsem_b = tuple(
        tuple(rest[3 * NBUF + 2 * b:3 * NBUF + 2 * b + 2]) for b in range(NBUF)
    )
    ssem_b = rest[5 * NBUF:6 * NBUF]

    wid = lax.axis_index("s") * NUM_CORES + lax.axis_index("c")
    base = wid * N

    pltpu.sync_copy(pos_hbm, pos_v)

    def gather_half(p, h):
        return (
            token_hbm.at[idx_b[p].at[h]],
            rows_b[p].at[pl.ds(h * SUBGATHER, SUBGATHER)],
            gsem_b[p][h],
        )

    def start_gathers(p):
        for h in range(SUBS):
            pltpu.async_copy(*gather_half(p, h))

    def wait_gather_half(p, h):
        pltpu.make_async_copy(*gather_half(p, h)).wait()

    def wait_idx(p):
        pltpu.make_async_copy(idx_hbm.at[base], idx_b[p], isem_b[p]).wait()

    def wait_store(p):
        pltpu.make_async_copy(
            rows_b[p], out_hbm.at[pl.ds(0, MAXLEN)], ssem_b[p]
        ).wait()

    def add_pos_rows(p, lo, n):
        def row_body(r, c2):
            for v in range(VECS_PER_ROW):
                sl = pl.ds(v * LANES, LANES)
                plsc.addupdate(rows_b[p].at[r, sl], pos_v[r, sl])
            return c2

        lax.fori_loop(lo, lo + n, row_body, 0, unroll=False)

    def store_rows(p, j, lo, n):
        pltpu.async_copy(
            rows_b[p].at[pl.ds(lo, n)],
            out_hbm.at[pl.ds((base + j) * MAXLEN + lo, n)],
            ssem_b[p],
        )

    pltpu.sync_copy(idx_hbm.at[base], idx_b[0])
    pltpu.sync_copy(idx_hbm.at[base + 1], idx_b[1])
    start_gathers(0)
    start_gathers(1)
    pltpu.async_copy(idx_hbm.at[base + 2], idx_b[2], isem_b[2])
    pltpu.async_copy(idx_hbm.at[base + 3], idx_b[3], isem_b[3])

    def group_body(i, carry):
        for p in range(NBUF):
            j = NBUF * i + p
            pf = p
            g2 = (p + 2) % NBUF

            @pl.when(j >= 2)
            def _():
                wait_store(g2)

            @pl.when(j + 2 < N)
            def _():
                wait_idx(g2)
                start_gathers(g2)

            for h in range(SUBS):
                wait_gather_half(p, h)
                if h == 0:
                    add_pos_rows(p, 0, STORE_LO)
                    store_rows(p, j, 0, STORE_LO)
                else:
                    @pl.when(j + NBUF < N)
                    def _():
                        pltpu.async_copy(
                            idx_hbm.at[base + j + NBUF], idx_b[pf], isem_b[pf]
                        )

                    add_pos_rows(p, STORE_LO, STORE_HI)
                    store_rows(p, j, STORE_LO, STORE_HI)
        return carry

    lax.fori_loop(0, N // NBUF, group_body, 0, unroll=False)
    wait_store((N - 2) % NBUF)
    wait_store((N - 1) % NBUF)


@functools.partial(jax.jit, static_argnames=())
def kernel(inputs, token_table, pos_table):
    idx = inputs.reshape(BATCH, SUBS, SUBGATHER).astype(jnp.int32)
    mesh = plsc.VectorSubcoreMesh(core_axis_name="c", subcore_axis_name="s")
    scratch = [pltpu.VMEM((MAXLEN, EMBED_DIM), jnp.float32)]
    scratch += [pltpu.VMEM((SUBS, SUBGATHER), jnp.int32)] * NBUF
    scratch += [pltpu.VMEM((MAXLEN, EMBED_DIM), jnp.float32)] * NBUF
    scratch += [pltpu.SemaphoreType.DMA] * NBUF
    scratch += [pltpu.SemaphoreType.DMA] * (2 * NBUF)
    scratch += [pltpu.SemaphoreType.DMA] * NBUF
    out = pl.kernel(
        _emb_kernel,
        mesh=mesh,
        out_type=jax.ShapeDtypeStruct((BATCH * MAXLEN, EMBED_DIM), jnp.float32),
        scratch_types=scratch,
    )(idx, token_table, pos_table)
    return out.reshape(BATCH, MAXLEN, EMBED_DIM)


# speedup vs baseline: 1.2821x; 1.0224x over previous
import functools

import jax
import jax.numpy as jnp
from jax import lax
from jax.experimental import pallas as pl
from jax.experimental.pallas import tpu as pltpu
from jax.experimental.pallas import tpu_sc as plsc

VOCAB_SIZE = 100000
EMBED_DIM = 128
MAXLEN = 200
BATCH = 1024

NUM_CORES = 2
NUM_SUBCORES = 16
NUM_WORKERS = NUM_CORES * NUM_SUBCORES

SUBGATHER = 100
SUBS = MAXLEN // SUBGATHER
N = BATCH // NUM_WORKERS
NBUF = 4
LANES = 16
VECS_PER_ROW = EMBED_DIM // LANES

STORE_LO = 96
STORE_HI = MAXLEN - STORE_LO


def _emb_kernel(idx_hbm, token_hbm, pos_hbm, out_hbm, pos_v, *rest):
    idx_b = rest[0:NBUF]
    rows_b = rest[NBUF:2 * NBUF]
    isem_b = rest[2 * NBUF:3 * NBUF]
    gsem_b = tuple(
        tuple(rest[3 * NBUF + 2 * b:3 * NBUF + 2 * b + 2]) for b in range(NBUF)
    )
    ssem_b = rest[5 * NBUF:6 * NBUF]
    psem = rest[6 * NBUF]

    wid = lax.axis_index("s") * NUM_CORES + lax.axis_index("c")
    base = wid * N

    pltpu.async_copy(pos_hbm, pos_v, psem)

    def gather_half(p, h):
        return (
            token_hbm.at[idx_b[p].at[h]],
            rows_b[p].at[pl.ds(h * SUBGATHER, SUBGATHER)],
            gsem_b[p][h],
        )

    def start_gathers(p):
        for h in range(SUBS):
            pltpu.async_copy(*gather_half(p, h))

    def wait_gather_half(p, h):
        pltpu.make_async_copy(*gather_half(p, h)).wait()

    def wait_idx(p):
        pltpu.make_async_copy(idx_hbm.at[base], idx_b[p], isem_b[p]).wait()

    def wait_store(p):
        pltpu.make_async_copy(
            rows_b[p], out_hbm.at[pl.ds(0, MAXLEN)], ssem_b[p]
        ).wait()

    def add_pos_rows(p, lo, n):
        def row_body(r, c2):
            for v in range(VECS_PER_ROW):
                sl = pl.ds(v * LANES, LANES)
                plsc.addupdate(rows_b[p].at[r, sl], pos_v[r, sl])
            return c2

        lax.fori_loop(lo, lo + n, row_body, 0, unroll=False)

    def store_rows(p, j, lo, n):
        pltpu.async_copy(
            rows_b[p].at[pl.ds(lo, n)],
            out_hbm.at[pl.ds((base + j) * MAXLEN + lo, n)],
            ssem_b[p],
        )

    pltpu.async_copy(idx_hbm.at[base], idx_b[0], isem_b[0])
    pltpu.async_copy(idx_hbm.at[base + 1], idx_b[1], isem_b[1])
    wait_idx(0)
    start_gathers(0)
    wait_idx(1)
    start_gathers(1)
    pltpu.async_copy(idx_hbm.at[base + 2], idx_b[2], isem_b[2])
    pltpu.async_copy(idx_hbm.at[base + 3], idx_b[3], isem_b[3])
    pltpu.make_async_copy(pos_hbm, pos_v, psem).wait()

    def group_body(i, carry):
        for p in range(NBUF):
            j = NBUF * i + p
            pf = p
            g2 = (p + 2) % NBUF

            @pl.when(j >= 2)
            def _():
                wait_store(g2)

            @pl.when(j + 2 < N)
            def _():
                wait_idx(g2)
                start_gathers(g2)

            for h in range(SUBS):
                wait_gather_half(p, h)
                if h == 0:
                    add_pos_rows(p, 0, STORE_LO)
                    store_rows(p, j, 0, STORE_LO)
                else:
                    @pl.when(j + NBUF < N)
                    def _():
                        pltpu.async_copy(
                            idx_hbm.at[base + j + NBUF], idx_b[pf], isem_b[pf]
                        )

                    add_pos_rows(p, STORE_LO, STORE_HI)
                    store_rows(p, j, STORE_LO, STORE_HI)
        return carry

    lax.fori_loop(0, N // NBUF, group_body, 0, unroll=False)
    wait_store((N - 2) % NBUF)
    wait_store((N - 1) % NBUF)


@functools.partial(jax.jit, static_argnames=())
def kernel(inputs, token_table, pos_table):
    idx = inputs.reshape(BATCH, SUBS, SUBGATHER).astype(jnp.int32)
    mesh = plsc.VectorSubcoreMesh(core_axis_name="c", subcore_axis_name="s")
    scratch = [pltpu.VMEM((MAXLEN, EMBED_DIM), jnp.float32)]
    scratch += [pltpu.VMEM((SUBS, SUBGATHER), jnp.int32)] * NBUF
    scratch += [pltpu.VMEM((MAXLEN, EMBED_DIM), jnp.float32)] * NBUF
    scratch += [pltpu.SemaphoreType.DMA] * NBUF
    scratch += [pltpu.SemaphoreType.DMA] * (2 * NBUF)
    scratch += [pltpu.SemaphoreType.DMA] * NBUF
    scratch += [pltpu.SemaphoreType.DMA]
    out = pl.kernel(
        _emb_kernel,
        mesh=mesh,
        out_type=jax.ShapeDtypeStruct((BATCH * MAXLEN, EMBED_DIM), jnp.float32),
        scratch_types=scratch,
    )(idx, token_table, pos_table)
    return out.reshape(BATCH, MAXLEN, EMBED_DIM)
